# Initial kernel scaffold; baseline (speedup 1.0000x reference)
import jax
import jax.numpy as jnp
from jax.experimental import pallas as pl


def kernel(x, kernel_emb, kernel_w):
    raise NotImplementedError("write your pallas kernel here")


# speedup vs baseline: 1.3286x; 1.3286x over previous
import functools

import jax
import jax.numpy as jnp
from jax import lax
from jax.experimental import pallas as pl
from jax.experimental.pallas import tpu as pltpu
from jax.experimental.pallas import tpu_sc as plsc

_B = 16384
_F = 26
_E = 16
_NC = 2
_NS = 16
_NW = _NC * _NS
_RPW = _B // _NW
_C = 64
_NCHUNK = _RPW // _C
_IPC = _C * _F
_G = 128
_NG = _IPC // _G

_mesh = plsc.VectorSubcoreMesh(core_axis_name="c", subcore_axis_name="s")


@functools.partial(
    pl.kernel,
    out_type=jax.ShapeDtypeStruct((_B,), jnp.float32),
    mesh=_mesh,
    compiler_params=pltpu.CompilerParams(
        needs_layout_passes=False, use_tc_tiling_on_sc=False
    ),
    scratch_types=[
        pltpu.VMEM((_IPC,), jnp.int32),
        pltpu.VMEM((_IPC, _E), jnp.float32),
        pltpu.VMEM((_IPC + 32,), jnp.float32),
        pltpu.VMEM((_RPW,), jnp.float32),
        pltpu.VMEM((256,), jnp.float32),
        pltpu.SemaphoreType.DMA,
        pltpu.SemaphoreType.DMA,
    ],
)
def _fm_sc(x_ref, emb_ref, w_ref, out_ref, idx_v, emb_v, w_v, out_v, mat_v, gsem, wsem):
    wid = lax.axis_index("s") * _NC + lax.axis_index("c")
    row0 = wid * _RPW
    lane = lax.iota(jnp.int32, 16)
    tail_mask = lane < (_F - 16)

    def chunk_body(g, carry):
        base_i = (row0 + g * _C) * _F
        pltpu.sync_copy(x_ref.at[pl.ds(base_i, _IPC)], idx_v)
        copies = []
        for j in range(_NG):
            sl = pl.ds(j * _G, _G)
            copies.append(pltpu.async_copy(emb_ref.at[idx_v.at[sl]], emb_v.at[sl], gsem))
            copies.append(pltpu.async_copy(w_ref.at[idx_v.at[sl]], w_v.at[sl], wsem))
        for cpy in copies:
            cpy.wait()

        def group_body(t, tc):
            def row_body(i, rc):
                r26 = (t * 16 + i) * _F
                e0 = emb_v[r26, :]
                s = e0
                q = e0 * e0
                for j in range(1, _F):
                    e = emb_v[r26 + j, :]
                    s = s + e
                    q = q + e * e
                w1 = w_v[pl.ds(r26, 16)]
                w2 = jnp.where(tail_mask, w_v[pl.ds(r26 + 16, 16)], 0.0)
                mat_v[pl.ds(i * 16, 16)] = 0.5 * (s * s - q) + w1 + w2
                return rc

            lax.fori_loop(0, 16, row_body, 0)
            acc = plsc.load_gather(mat_v, [lane * 16])
            for d in range(1, 16):
                acc = acc + plsc.load_gather(mat_v, [lane * 16 + d])
            out_v[pl.ds(g * _C + t * 16, 16)] = acc
            return tc

        return lax.fori_loop(0, _C // 16, group_body, carry)

    lax.fori_loop(0, _NCHUNK, chunk_body, 0)
    pltpu.sync_copy(out_v, out_ref.at[pl.ds(row0, _RPW)])


def kernel(x, kernel_emb, kernel_w):
    out = _fm_sc(x.reshape(-1), kernel_emb, kernel_w.reshape(-1))
    return out.reshape(_B, 1)


# speedup vs baseline: 1.4737x; 1.1092x over previous
import functools

import jax
import jax.numpy as jnp
from jax import lax
from jax.experimental import pallas as pl
from jax.experimental.pallas import tpu as pltpu
from jax.experimental.pallas import tpu_sc as plsc

_B = 16384
_F = 26
_E = 16
_NC = 2
_NS = 16
_NW = _NC * _NS
_RPW = _B // _NW
_C = 64
_NCHUNK = _RPW // _C
_IPC = _C * _F
_G = 128
_NG = _IPC // _G

_mesh = plsc.VectorSubcoreMesh(core_axis_name="c", subcore_axis_name="s")

_NB = 7813
_NBF = _NB - 1
_SPW = 245
_DEPTH = 4


@functools.partial(
    pl.kernel,
    out_type=jax.ShapeDtypeStruct((16000000,), jnp.float32),
    mesh=_mesh,
    compiler_params=pltpu.CompilerParams(
        needs_layout_passes=False, use_tc_tiling_on_sc=True
    ),
    scratch_types=[
        pltpu.VMEM((_DEPTH, 16, 128), jnp.float32),
        pltpu.VMEM((2, 2048), jnp.float32),
        pltpu.SemaphoreType.DMA,
        pltpu.SemaphoreType.DMA,
    ],
)
def _detile_sc(emb_t_ref, out_ref, ibuf, obuf, isem, osem):
    wid = lax.axis_index("s") * _NC + lax.axis_index("c")
    lane = lax.iota(jnp.int32, 16)

    def in_copy(step, slot):
        c0 = (wid + step * _NW) * 128
        return pltpu.make_async_copy(
            emb_t_ref.at[:, pl.ds(c0, 128)], ibuf.at[slot], isem
        )

    for k in range(_DEPTH):
        @pl.when(wid + k * _NW < _NBF)
        def _():
            in_copy(k, k).start()

    def step_body(step, carry):
        tj = wid + step * _NW
        slot = lax.rem(step, _DEPTH)
        oslot = lax.rem(step, 2)

        @pl.when(tj < _NBF)
        def _():
            in_copy(step, slot).wait()
            @pl.when(step >= 2)
            def _():
                pltpu.make_async_copy(
                    obuf.at[oslot], out_ref.at[pl.ds(0, 2048)], osem
                ).wait()
            for rr in range(128):
                v = plsc.load_gather(ibuf.at[slot], [lane, jnp.full((16,), rr, jnp.int32)])
                obuf[oslot, pl.ds(rr * 16, 16)] = v
            pltpu.async_copy(obuf.at[oslot], out_ref.at[pl.ds(tj * 2048, 2048)], osem)
            @pl.when(step + _DEPTH < _SPW)
            def _():
                @pl.when(tj + _DEPTH * _NW < _NBF)
                def _():
                    in_copy(step + _DEPTH, slot).start()
        return carry

    lax.fori_loop(0, _SPW, step_body, 0)

    nvalid = lax.div(_NBF - wid + _NW - 1, _NW)
    for t in range(2):
        s = nvalid - 2 + t
        @pl.when(s >= 0)
        def _():
            pltpu.make_async_copy(
                obuf.at[lax.rem(s, 2)], out_ref.at[pl.ds(0, 2048)], osem
            ).wait()


@functools.partial(
    pl.kernel,
    out_type=jax.ShapeDtypeStruct((_B,), jnp.float32),
    mesh=_mesh,
    compiler_params=pltpu.CompilerParams(
        needs_layout_passes=False, use_tc_tiling_on_sc=False
    ),
    scratch_types=[
        pltpu.VMEM((_IPC,), jnp.int32),
        pltpu.VMEM((_IPC, _E), jnp.float32),
        pltpu.VMEM((_IPC + 32,), jnp.float32),
        pltpu.VMEM((_RPW,), jnp.float32),
        pltpu.VMEM((256,), jnp.float32),
        pltpu.SemaphoreType.DMA,
        pltpu.SemaphoreType.DMA,
    ],
)
def _fm_sc(x_ref, emb_ref, w_ref, out_ref, idx_v, emb_v, w_v, out_v, mat_v, gsem, wsem):
    wid = lax.axis_index("s") * _NC + lax.axis_index("c")
    row0 = wid * _RPW
    lane = lax.iota(jnp.int32, 16)
    tail_mask = lane < (_F - 16)

    def chunk_body(g, carry):
        base_i = (row0 + g * _C) * _F
        pltpu.sync_copy(x_ref.at[pl.ds(base_i, _IPC)], idx_v)
        copies = []
        for j in range(_NG):
            sl = pl.ds(j * _G, _G)
            copies.append(pltpu.async_copy(emb_ref.at[idx_v.at[sl]], emb_v.at[sl], gsem))
            copies.append(pltpu.async_copy(w_ref.at[idx_v.at[sl]], w_v.at[sl], wsem))
        for cpy in copies:
            cpy.wait()

        def group_body(t, tc):
            def row_body(i, rc):
                r26 = (t * 16 + i) * _F
                e0 = emb_v[r26, :]
                s = e0
                q = e0 * e0
                for j in range(1, _F):
                    e = emb_v[r26 + j, :]
                    s = s + e
                    q = q + e * e
                w1 = w_v[pl.ds(r26, 16)]
                w2 = jnp.where(tail_mask, w_v[pl.ds(r26 + 16, 16)], 0.0)
                mat_v[pl.ds(i * 16, 16)] = 0.5 * (s * s - q) + w1 + w2
                return rc

            lax.fori_loop(0, 16, row_body, 0)
            acc = plsc.load_gather(mat_v, [lane * 16])
            for d in range(1, 16):
                acc = acc + plsc.load_gather(mat_v, [lane * 16 + d])
            out_v[pl.ds(g * _C + t * 16, 16)] = acc
            return tc

        return lax.fori_loop(0, _C // 16, group_body, carry)

    lax.fori_loop(0, _NCHUNK, chunk_body, 0)
    pltpu.sync_copy(out_v, out_ref.at[pl.ds(row0, _RPW)])


def kernel(x, kernel_emb, kernel_w):
    emb_flat = _detile_sc(kernel_emb.T)
    tail = kernel_emb[_NBF * 128 :].reshape(-1)
    emb_flat = jax.lax.dynamic_update_slice(emb_flat, tail, (_NBF * 128 * _E,))
    table = emb_flat.reshape(1000000, _E)
    out = _fm_sc(x.reshape(-1), table, kernel_w.reshape(-1))
    return out.reshape(_B, 1)


# speedup vs baseline: 2.1718x; 1.4737x over previous
import functools

import jax
import jax.numpy as jnp
from jax import lax
from jax.experimental import pallas as pl
from jax.experimental.pallas import tpu as pltpu
from jax.experimental.pallas import tpu_sc as plsc

_B = 16384
_F = 26
_E = 16
_NC = 2
_NS = 16
_NW = _NC * _NS
_RPW = _B // _NW
_C = 64
_NCHUNK = _RPW // _C
_IPC = _C * _F
_G = 128
_NG = _IPC // _G

_mesh = plsc.VectorSubcoreMesh(core_axis_name="c", subcore_axis_name="s")

_NB = 7813
_NBF = _NB - 1
_SPW = 245
_DEPTH = 4


@functools.partial(
    pl.kernel,
    out_type=jax.ShapeDtypeStruct((16000000,), jnp.float32),
    mesh=_mesh,
    compiler_params=pltpu.CompilerParams(
        needs_layout_passes=False, use_tc_tiling_on_sc=True
    ),
    scratch_types=[
        pltpu.VMEM((_DEPTH, 16, 129), jnp.float32),
        pltpu.VMEM((2, 2048), jnp.float32),
        pltpu.SemaphoreType.DMA,
        pltpu.SemaphoreType.DMA,
    ],
)
def _detile_sc(emb_t_ref, out_ref, ibuf, obuf, isem, osem):
    wid = lax.axis_index("s") * _NC + lax.axis_index("c")
    lane = lax.iota(jnp.int32, 16)

    def in_copy(step, slot):
        c0 = (wid + step * _NW) * 128
        return pltpu.make_async_copy(
            emb_t_ref.at[:, pl.ds(c0, 128)], ibuf.at[slot, :, pl.ds(0, 128)], isem
        )

    for k in range(_DEPTH):
        @pl.when(wid + k * _NW < _NBF)
        def _():
            in_copy(k, k).start()

    def step_body(step, carry):
        tj = wid + step * _NW
        slot = lax.rem(step, _DEPTH)
        oslot = lax.rem(step, 2)

        @pl.when(tj < _NBF)
        def _():
            in_copy(step, slot).wait()
            @pl.when(step >= 2)
            def _():
                pltpu.make_async_copy(
                    obuf.at[oslot], out_ref.at[pl.ds(0, 2048)], osem
                ).wait()
            for g in range(8):
                vals = [
                    plsc.load_gather(
                        ibuf.at[slot],
                        [lane, jnp.full((16,), g * 16 + l, jnp.int32)],
                    )
                    for l in range(16)
                ]
                for l in range(16):
                    obuf[oslot, pl.ds((g * 16 + l) * 16, 16)] = vals[l]
            pltpu.async_copy(obuf.at[oslot], out_ref.at[pl.ds(tj * 2048, 2048)], osem)
            @pl.when(step + _DEPTH < _SPW)
            def _():
                @pl.when(tj + _DEPTH * _NW < _NBF)
                def _():
                    in_copy(step + _DEPTH, slot).start()
        return carry

    lax.fori_loop(0, _SPW, step_body, 0)

    nvalid = lax.div(_NBF - wid + _NW - 1, _NW)
    for t in range(2):
        s = nvalid - 2 + t
        @pl.when(s >= 0)
        def _():
            pltpu.make_async_copy(
                obuf.at[lax.rem(s, 2)], out_ref.at[pl.ds(0, 2048)], osem
            ).wait()


@functools.partial(
    pl.kernel,
    out_type=jax.ShapeDtypeStruct((_B,), jnp.float32),
    mesh=_mesh,
    compiler_params=pltpu.CompilerParams(
        needs_layout_passes=False, use_tc_tiling_on_sc=False
    ),
    scratch_types=[
        pltpu.VMEM((_IPC,), jnp.int32),
        pltpu.VMEM((_IPC, _E), jnp.float32),
        pltpu.VMEM((_IPC + 32,), jnp.float32),
        pltpu.VMEM((_RPW,), jnp.float32),
        pltpu.VMEM((256,), jnp.float32),
        pltpu.SemaphoreType.DMA,
        pltpu.SemaphoreType.DMA,
    ],
)
def _fm_sc(x_ref, emb_ref, w_ref, out_ref, idx_v, emb_v, w_v, out_v, mat_v, gsem, wsem):
    wid = lax.axis_index("s") * _NC + lax.axis_index("c")
    row0 = wid * _RPW
    lane = lax.iota(jnp.int32, 16)
    tail_mask = lane < (_F - 16)

    def chunk_body(g, carry):
        base_i = (row0 + g * _C) * _F
        pltpu.sync_copy(x_ref.at[pl.ds(base_i, _IPC)], idx_v)
        copies = []
        for j in range(_NG):
            sl = pl.ds(j * _G, _G)
            copies.append(pltpu.async_copy(emb_ref.at[idx_v.at[sl]], emb_v.at[sl], gsem))
            copies.append(pltpu.async_copy(w_ref.at[idx_v.at[sl]], w_v.at[sl], wsem))
        for cpy in copies:
            cpy.wait()

        def group_body(t, tc):
            def row_body(i, rc):
                r26 = (t * 16 + i) * _F
                e0 = emb_v[r26, :]
                s = e0
                q = e0 * e0
                for j in range(1, _F):
                    e = emb_v[r26 + j, :]
                    s = s + e
                    q = q + e * e
                w1 = w_v[pl.ds(r26, 16)]
                w2 = jnp.where(tail_mask, w_v[pl.ds(r26 + 16, 16)], 0.0)
                mat_v[pl.ds(i * 16, 16)] = 0.5 * (s * s - q) + w1 + w2
                return rc

            lax.fori_loop(0, 16, row_body, 0)
            acc = plsc.load_gather(mat_v, [lane * 16])
            for d in range(1, 16):
                acc = acc + plsc.load_gather(mat_v, [lane * 16 + d])
            out_v[pl.ds(g * _C + t * 16, 16)] = acc
            return tc

        return lax.fori_loop(0, _C // 16, group_body, carry)

    lax.fori_loop(0, _NCHUNK, chunk_body, 0)
    pltpu.sync_copy(out_v, out_ref.at[pl.ds(row0, _RPW)])


def kernel(x, kernel_emb, kernel_w):
    emb_flat = _detile_sc(kernel_emb.T)
    tail = kernel_emb[_NBF * 128 :].reshape(-1)
    emb_flat = jax.lax.dynamic_update_slice(emb_flat, tail, (_NBF * 128 * _E,))
    table = emb_flat.reshape(1000000, _E)
    out = _fm_sc(x.reshape(-1), table, kernel_w.reshape(-1))
    return out.reshape(_B, 1)


# speedup vs baseline: 2.1821x; 1.0048x over previous
import functools

import jax
import jax.numpy as jnp
from jax import lax
from jax.experimental import pallas as pl
from jax.experimental.pallas import tpu as pltpu
from jax.experimental.pallas import tpu_sc as plsc

_B = 16384
_F = 26
_E = 16
_NC = 2
_NS = 16
_NW = _NC * _NS
_RPW = _B // _NW
_C = 64
_NCHUNK = _RPW // _C
_IPC = _C * _F
_G = 128
_NG = _IPC // _G

_mesh = plsc.VectorSubcoreMesh(core_axis_name="c", subcore_axis_name="s")

_NB = 7813
_NBF = _NB - 1
_SPW = 245
_DEPTH = 4


@functools.partial(
    pl.kernel,
    out_type=jax.ShapeDtypeStruct((16000000,), jnp.float32),
    mesh=_mesh,
    compiler_params=pltpu.CompilerParams(
        needs_layout_passes=False, use_tc_tiling_on_sc=True
    ),
    scratch_types=[
        pltpu.VMEM((_DEPTH, 16, 136), jnp.float32),
        pltpu.VMEM((2, 2048), jnp.float32),
        pltpu.SemaphoreType.DMA,
        pltpu.SemaphoreType.DMA,
    ],
)
def _detile_sc(emb_t_ref, out_ref, ibuf, obuf, isem, osem):
    wid = lax.axis_index("s") * _NC + lax.axis_index("c")
    lane = lax.iota(jnp.int32, 16)

    def in_copy(step, slot):
        c0 = (wid + step * _NW) * 128
        return pltpu.make_async_copy(
            emb_t_ref.at[:, pl.ds(c0, 128)], ibuf.at[slot, :, pl.ds(0, 128)], isem
        )

    for k in range(_DEPTH):
        @pl.when(wid + k * _NW < _NBF)
        def _():
            in_copy(k, k).start()

    def step_body(step, carry):
        tj = wid + step * _NW
        slot = lax.rem(step, _DEPTH)
        oslot = lax.rem(step, 2)

        @pl.when(tj < _NBF)
        def _():
            in_copy(step, slot).wait()
            @pl.when(step >= 2)
            def _():
                pltpu.make_async_copy(
                    obuf.at[oslot], out_ref.at[pl.ds(0, 2048)], osem
                ).wait()
            for g in range(8):
                vals = [
                    plsc.load_gather(
                        ibuf.at[slot],
                        [lane, jnp.full((16,), g * 16 + l, jnp.int32)],
                    )
                    for l in range(16)
                ]
                for l in range(16):
                    obuf[oslot, pl.ds((g * 16 + l) * 16, 16)] = vals[l]
            pltpu.async_copy(obuf.at[oslot], out_ref.at[pl.ds(tj * 2048, 2048)], osem)
            @pl.when(step + _DEPTH < _SPW)
            def _():
                @pl.when(tj + _DEPTH * _NW < _NBF)
                def _():
                    in_copy(step + _DEPTH, slot).start()
        return carry

    lax.fori_loop(0, _SPW, step_body, 0)

    nvalid = lax.div(_NBF - wid + _NW - 1, _NW)
    for t in range(2):
        s = nvalid - 2 + t
        @pl.when(s >= 0)
        def _():
            pltpu.make_async_copy(
                obuf.at[lax.rem(s, 2)], out_ref.at[pl.ds(0, 2048)], osem
            ).wait()


@functools.partial(
    pl.kernel,
    out_type=jax.ShapeDtypeStruct((_B,), jnp.float32),
    mesh=_mesh,
    compiler_params=pltpu.CompilerParams(
        needs_layout_passes=False, use_tc_tiling_on_sc=False
    ),
    scratch_types=[
        pltpu.VMEM((_IPC,), jnp.int32),
        pltpu.VMEM((_IPC, _E), jnp.float32),
        pltpu.VMEM((_IPC + 32,), jnp.float32),
        pltpu.VMEM((_RPW,), jnp.float32),
        pltpu.VMEM((256,), jnp.float32),
        pltpu.SemaphoreType.DMA,
        pltpu.SemaphoreType.DMA,
    ],
)
def _fm_sc(x_ref, emb_ref, w_ref, out_ref, idx_v, emb_v, w_v, out_v, mat_v, gsem, wsem):
    wid = lax.axis_index("s") * _NC + lax.axis_index("c")
    row0 = wid * _RPW
    lane = lax.iota(jnp.int32, 16)
    tail_mask = lane < (_F - 16)

    def chunk_body(g, carry):
        base_i = (row0 + g * _C) * _F
        pltpu.sync_copy(x_ref.at[pl.ds(base_i, _IPC)], idx_v)
        copies = []
        for j in range(_NG):
            sl = pl.ds(j * _G, _G)
            copies.append(pltpu.async_copy(emb_ref.at[idx_v.at[sl]], emb_v.at[sl], gsem))
            copies.append(pltpu.async_copy(w_ref.at[idx_v.at[sl]], w_v.at[sl], wsem))
        for cpy in copies:
            cpy.wait()

        def group_body(t, tc):
            def row_body(i, rc):
                r26 = (t * 16 + i) * _F
                e0 = emb_v[r26, :]
                s = e0
                q = e0 * e0
                for j in range(1, _F):
                    e = emb_v[r26 + j, :]
                    s = s + e
                    q = q + e * e
                w1 = w_v[pl.ds(r26, 16)]
                w2 = jnp.where(tail_mask, w_v[pl.ds(r26 + 16, 16)], 0.0)
                mat_v[pl.ds(i * 16, 16)] = 0.5 * (s * s - q) + w1 + w2
                return rc

            lax.fori_loop(0, 16, row_body, 0)
            acc = plsc.load_gather(mat_v, [lane * 16])
            for d in range(1, 16):
                acc = acc + plsc.load_gather(mat_v, [lane * 16 + d])
            out_v[pl.ds(g * _C + t * 16, 16)] = acc
            return tc

        return lax.fori_loop(0, _C // 16, group_body, carry)

    lax.fori_loop(0, _NCHUNK, chunk_body, 0)
    pltpu.sync_copy(out_v, out_ref.at[pl.ds(row0, _RPW)])


def kernel(x, kernel_emb, kernel_w):
    emb_flat = _detile_sc(kernel_emb.T)
    tail = kernel_emb[_NBF * 128 :].reshape(-1)
    emb_flat = jax.lax.dynamic_update_slice(emb_flat, tail, (_NBF * 128 * _E,))
    table = emb_flat.reshape(1000000, _E)
    out = _fm_sc(x.reshape(-1), table, kernel_w.reshape(-1))
    return out.reshape(_B, 1)


# speedup vs baseline: 4.1450x; 1.8995x over previous
import functools

import jax
import jax.numpy as jnp
from jax import lax
from jax.experimental import pallas as pl
from jax.experimental.pallas import tpu as pltpu
from jax.experimental.pallas import tpu_sc as plsc

_B = 16384
_F = 26
_E = 16
_NC = 2
_NS = 16
_NW = _NC * _NS
_RPW = _B // _NW
_C = 64
_NCHUNK = _RPW // _C
_IPC = _C * _F
_G = 128
_NG = _IPC // _G

_mesh = plsc.VectorSubcoreMesh(core_axis_name="c", subcore_axis_name="s")

_NB = 7813
_NBF = _NB - 1
_SPW = 245
_DEPTH = 4


@functools.partial(
    pl.kernel,
    out_type=jax.ShapeDtypeStruct((16000000,), jnp.float32),
    mesh=_mesh,
    compiler_params=pltpu.CompilerParams(
        needs_layout_passes=False, use_tc_tiling_on_sc=True
    ),
    scratch_types=[
        pltpu.VMEM((_DEPTH, 16, 128), jnp.float32),
        pltpu.VMEM((2, 2048), jnp.float32),
        pltpu.SemaphoreType.DMA,
        pltpu.SemaphoreType.DMA,
    ],
)
def _detile_sc(emb_t_ref, out_ref, ibuf, obuf, isem, osem):
    wid = lax.axis_index("s") * _NC + lax.axis_index("c")
    lane = lax.iota(jnp.int32, 16)
    _dn = lax.GatherDimensionNumbers(
        offset_dims=(), collapsed_slice_dims=(0,), start_index_map=(0,)
    )
    perms = {s: (lane ^ s)[:, None] for s in (1, 2, 4, 8)}
    masks = {s: (lane & s) == 0 for s in (1, 2, 4, 8)}

    def _shuf(v, s):
        return lax.gather(
            v, perms[s], _dn, (1,), mode=lax.GatherScatterMode.PROMISE_IN_BOUNDS
        )

    def _transpose16(t):
        for s in (8, 4, 2, 1):
            m = masks[s]
            for i in range(16):
                if i & s:
                    continue
                j = i | s
                a, b = t[i], t[j]
                sh = _shuf(jnp.where(m, b, a), s)
                t[i] = jnp.where(m, a, sh)
                t[j] = jnp.where(m, sh, b)
        return t

    def in_copy(step, slot):
        c0 = (wid + step * _NW) * 128
        return pltpu.make_async_copy(
            emb_t_ref.at[:, pl.ds(c0, 128)], ibuf.at[slot, :, pl.ds(0, 128)], isem
        )

    for k in range(_DEPTH):
        @pl.when(wid + k * _NW < _NBF)
        def _():
            in_copy(k, k).start()

    def step_body(step, carry):
        tj = wid + step * _NW
        slot = lax.rem(step, _DEPTH)
        oslot = lax.rem(step, 2)

        @pl.when(tj < _NBF)
        def _():
            in_copy(step, slot).wait()
            @pl.when(step >= 2)
            def _():
                pltpu.make_async_copy(
                    obuf.at[oslot], out_ref.at[pl.ds(0, 2048)], osem
                ).wait()
            for g in range(8):
                t = [ibuf[slot, c, pl.ds(g * 16, 16)] for c in range(16)]
                t = _transpose16(t)
                for r in range(16):
                    obuf[oslot, pl.ds((g * 16 + r) * 16, 16)] = t[r]
            pltpu.async_copy(obuf.at[oslot], out_ref.at[pl.ds(tj * 2048, 2048)], osem)
            @pl.when(step + _DEPTH < _SPW)
            def _():
                @pl.when(tj + _DEPTH * _NW < _NBF)
                def _():
                    in_copy(step + _DEPTH, slot).start()
        return carry

    lax.fori_loop(0, _SPW, step_body, 0)

    nvalid = lax.div(_NBF - wid + _NW - 1, _NW)
    for t in range(2):
        s = nvalid - 2 + t
        @pl.when(s >= 0)
        def _():
            pltpu.make_async_copy(
                obuf.at[lax.rem(s, 2)], out_ref.at[pl.ds(0, 2048)], osem
            ).wait()


@functools.partial(
    pl.kernel,
    out_type=jax.ShapeDtypeStruct((_B,), jnp.float32),
    mesh=_mesh,
    compiler_params=pltpu.CompilerParams(
        needs_layout_passes=False, use_tc_tiling_on_sc=False
    ),
    scratch_types=[
        pltpu.VMEM((_IPC,), jnp.int32),
        pltpu.VMEM((_IPC, _E), jnp.float32),
        pltpu.VMEM((_IPC + 32,), jnp.float32),
        pltpu.VMEM((_RPW,), jnp.float32),
        pltpu.VMEM((256,), jnp.float32),
        pltpu.SemaphoreType.DMA,
        pltpu.SemaphoreType.DMA,
    ],
)
def _fm_sc(x_ref, emb_ref, w_ref, out_ref, idx_v, emb_v, w_v, out_v, mat_v, gsem, wsem):
    wid = lax.axis_index("s") * _NC + lax.axis_index("c")
    row0 = wid * _RPW
    lane = lax.iota(jnp.int32, 16)
    tail_mask = lane < (_F - 16)

    def chunk_body(g, carry):
        base_i = (row0 + g * _C) * _F
        pltpu.sync_copy(x_ref.at[pl.ds(base_i, _IPC)], idx_v)
        copies = []
        for j in range(_NG):
            sl = pl.ds(j * _G, _G)
            copies.append(pltpu.async_copy(emb_ref.at[idx_v.at[sl]], emb_v.at[sl], gsem))
            copies.append(pltpu.async_copy(w_ref.at[idx_v.at[sl]], w_v.at[sl], wsem))
        for cpy in copies:
            cpy.wait()

        def group_body(t, tc):
            def row_body(i, rc):
                r26 = (t * 16 + i) * _F
                e0 = emb_v[r26, :]
                s = e0
                q = e0 * e0
                for j in range(1, _F):
                    e = emb_v[r26 + j, :]
                    s = s + e
                    q = q + e * e
                w1 = w_v[pl.ds(r26, 16)]
                w2 = jnp.where(tail_mask, w_v[pl.ds(r26 + 16, 16)], 0.0)
                mat_v[pl.ds(i * 16, 16)] = 0.5 * (s * s - q) + w1 + w2
                return rc

            lax.fori_loop(0, 16, row_body, 0)
            acc = plsc.load_gather(mat_v, [lane * 16])
            for d in range(1, 16):
                acc = acc + plsc.load_gather(mat_v, [lane * 16 + d])
            out_v[pl.ds(g * _C + t * 16, 16)] = acc
            return tc

        return lax.fori_loop(0, _C // 16, group_body, carry)

    lax.fori_loop(0, _NCHUNK, chunk_body, 0)
    pltpu.sync_copy(out_v, out_ref.at[pl.ds(row0, _RPW)])


def kernel(x, kernel_emb, kernel_w):
    emb_flat = _detile_sc(kernel_emb.T)
    tail = kernel_emb[_NBF * 128 :].reshape(-1)
    emb_flat = jax.lax.dynamic_update_slice(emb_flat, tail, (_NBF * 128 * _E,))
    table = emb_flat.reshape(1000000, _E)
    out = _fm_sc(x.reshape(-1), table, kernel_w.reshape(-1))
    return out.reshape(_B, 1)


# speedup vs baseline: 4.4083x; 1.0635x over previous
import functools

import jax
import jax.numpy as jnp
from jax import lax
from jax.experimental import pallas as pl
from jax.experimental.pallas import tpu as pltpu
from jax.experimental.pallas import tpu_sc as plsc

_B = 16384
_F = 26
_E = 16
_NC = 2
_NS = 16
_NW = _NC * _NS
_RPW = _B // _NW
_C = 64
_NCHUNK = _RPW // _C
_IPC = _C * _F
_G = 128
_NG = _IPC // _G

_mesh = plsc.VectorSubcoreMesh(core_axis_name="c", subcore_axis_name="s")

_NB = 7813
_NBF = _NB - 1
_SPW = 245
_DEPTH = 4


@functools.partial(
    pl.kernel,
    out_type=jax.ShapeDtypeStruct((16000000,), jnp.float32),
    mesh=_mesh,
    compiler_params=pltpu.CompilerParams(
        needs_layout_passes=False, use_tc_tiling_on_sc=True
    ),
    scratch_types=[
        pltpu.VMEM((_DEPTH, 16, 128), jnp.float32),
        pltpu.VMEM((2, 2048), jnp.float32),
        pltpu.SemaphoreType.DMA,
        pltpu.SemaphoreType.DMA,
    ],
)
def _detile_sc(emb_t_ref, out_ref, ibuf, obuf, isem, osem):
    wid = lax.axis_index("s") * _NC + lax.axis_index("c")
    lane = lax.iota(jnp.int32, 16)
    _dn = lax.GatherDimensionNumbers(
        offset_dims=(), collapsed_slice_dims=(0,), start_index_map=(0,)
    )
    perms = {s: (lane ^ s)[:, None] for s in (1, 2, 4, 8)}
    masks = {s: (lane & s) == 0 for s in (1, 2, 4, 8)}

    def _shuf(v, s):
        return lax.gather(
            v, perms[s], _dn, (1,), mode=lax.GatherScatterMode.PROMISE_IN_BOUNDS
        )

    def _transpose16(t):
        for s in (8, 4, 2, 1):
            m = masks[s]
            for i in range(16):
                if i & s:
                    continue
                j = i | s
                a, b = t[i], t[j]
                sh = _shuf(jnp.where(m, b, a), s)
                t[i] = jnp.where(m, a, sh)
                t[j] = jnp.where(m, sh, b)
        return t

    def in_copy(step, slot):
        c0 = (wid + step * _NW) * 128
        return pltpu.make_async_copy(
            emb_t_ref.at[:, pl.ds(c0, 128)], ibuf.at[slot, :, pl.ds(0, 128)], isem
        )

    for k in range(_DEPTH):
        @pl.when(wid + k * _NW < _NBF)
        def _():
            in_copy(k, k).start()

    def step_body(step, carry):
        tj = wid + step * _NW
        slot = lax.rem(step, _DEPTH)
        oslot = lax.rem(step, 2)

        @pl.when(tj < _NBF)
        def _():
            in_copy(step, slot).wait()
            @pl.when(step >= 2)
            def _():
                pltpu.make_async_copy(
                    obuf.at[oslot], out_ref.at[pl.ds(0, 2048)], osem
                ).wait()
            for g in range(8):
                t = [ibuf[slot, c, pl.ds(g * 16, 16)] for c in range(16)]
                t = _transpose16(t)
                for r in range(16):
                    obuf[oslot, pl.ds((g * 16 + r) * 16, 16)] = t[r]
            pltpu.async_copy(obuf.at[oslot], out_ref.at[pl.ds(tj * 2048, 2048)], osem)
            @pl.when(step + _DEPTH < _SPW)
            def _():
                @pl.when(tj + _DEPTH * _NW < _NBF)
                def _():
                    in_copy(step + _DEPTH, slot).start()
        return carry

    lax.fori_loop(0, _SPW, step_body, 0)

    nvalid = lax.div(_NBF - wid + _NW - 1, _NW)
    for t in range(2):
        s = nvalid - 2 + t
        @pl.when(s >= 0)
        def _():
            pltpu.make_async_copy(
                obuf.at[lax.rem(s, 2)], out_ref.at[pl.ds(0, 2048)], osem
            ).wait()


@functools.partial(
    pl.kernel,
    out_type=jax.ShapeDtypeStruct((_B,), jnp.float32),
    mesh=_mesh,
    compiler_params=pltpu.CompilerParams(
        needs_layout_passes=False, use_tc_tiling_on_sc=False
    ),
    scratch_types=[
        pltpu.VMEM((2, _IPC), jnp.int32),
        pltpu.VMEM((2, _IPC, _E), jnp.float32),
        pltpu.VMEM((2, _IPC + 32), jnp.float32),
        pltpu.VMEM((_RPW,), jnp.float32),
        pltpu.VMEM((256,), jnp.float32),
        pltpu.SemaphoreType.DMA,
        pltpu.SemaphoreType.DMA,
    ],
)
def _fm_sc(x_ref, emb_ref, w_ref, out_ref, idx_v, emb_v, w_v, out_v, mat_v, gsem, wsem):
    wid = lax.axis_index("s") * _NC + lax.axis_index("c")
    row0 = wid * _RPW
    lane = lax.iota(jnp.int32, 16)
    tail_mask = lane < (_F - 16)

    def stage_idx(g, slot):
        base_i = (row0 + g * _C) * _F
        pltpu.sync_copy(x_ref.at[pl.ds(base_i, _IPC)], idx_v.at[slot])

    def fire(slot):
        for j in range(_NG):
            sl = pl.ds(j * _G, _G)
            pltpu.async_copy(emb_ref.at[idx_v.at[slot, sl]], emb_v.at[slot, sl], gsem)
            pltpu.async_copy(w_ref.at[idx_v.at[slot, sl]], w_v.at[slot, sl], wsem)

    stage_idx(0, 0)
    fire(0)
    stage_idx(1, 1)

    def chunk_body(g, carry):
        slot = lax.rem(g, 2)
        nslot = lax.rem(g + 1, 2)

        @pl.when(g + 1 < _NCHUNK)
        def _():
            fire(nslot)
        pltpu.make_async_copy(
            emb_ref.at[pl.ds(0, _IPC)], emb_v.at[slot], gsem
        ).wait()
        pltpu.make_async_copy(
            w_ref.at[pl.ds(0, _IPC)], w_v.at[slot, pl.ds(0, _IPC)], wsem
        ).wait()

        @pl.when(g + 2 < _NCHUNK)
        def _():
            stage_idx(g + 2, slot)

        def group_body(t, tc):
            def row_body(i, rc):
                r26 = (t * 16 + i) * _F
                e0 = emb_v[slot, r26, :]
                s = e0
                q = e0 * e0
                for j in range(1, _F):
                    e = emb_v[slot, r26 + j, :]
                    s = s + e
                    q = q + e * e
                w1 = w_v[slot, pl.ds(r26, 16)]
                w2 = jnp.where(tail_mask, w_v[slot, pl.ds(r26 + 16, 16)], 0.0)
                mat_v[pl.ds(i * 16, 16)] = 0.5 * (s * s - q) + w1 + w2
                return rc

            lax.fori_loop(0, 16, row_body, 0)
            acc = plsc.load_gather(mat_v, [lane * 16])
            for d in range(1, 16):
                acc = acc + plsc.load_gather(mat_v, [lane * 16 + d])
            out_v[pl.ds(g * _C + t * 16, 16)] = acc
            return tc

        return lax.fori_loop(0, _C // 16, group_body, carry)

    lax.fori_loop(0, _NCHUNK, chunk_body, 0)
    pltpu.sync_copy(out_v, out_ref.at[pl.ds(row0, _RPW)])


def kernel(x, kernel_emb, kernel_w):
    emb_flat = _detile_sc(kernel_emb.T)
    tail = kernel_emb[_NBF * 128 :].reshape(-1)
    emb_flat = jax.lax.dynamic_update_slice(emb_flat, tail, (_NBF * 128 * _E,))
    table = emb_flat.reshape(1000000, _E)
    out = _fm_sc(x.reshape(-1), table, kernel_w.reshape(-1))
    return out.reshape(_B, 1)
